# Initial kernel scaffold; baseline (speedup 1.0000x reference)
import jax
import jax.numpy as jnp
from jax.experimental import pallas as pl


def kernel(idx, embn, embhe, W1, b1, W2, b2):
    raise NotImplementedError("write your pallas kernel here")


# speedup vs baseline: 7.2964x; 7.2964x over previous
import functools

import jax
import jax.numpy as jnp
from jax import lax
from jax.experimental import pallas as pl
from jax.experimental.pallas import tpu as pltpu
from jax.experimental.pallas import tpu_sc as plsc

_ALPHA = 3.0
_K = 20


def _gather_rows_sc(table, idx):
    info = plsc.get_sparse_core_info()
    nc, ns = info.num_cores, info.num_subcores
    nw = nc * ns
    b, d = idx.shape[0], table.shape[1]
    b_per_w = b // nw
    mesh = plsc.VectorSubcoreMesh(core_axis_name="c", subcore_axis_name="s")

    @functools.partial(
        pl.kernel,
        mesh=mesh,
        compiler_params=pltpu.CompilerParams(use_tc_tiling_on_sc=False),
        out_type=jax.ShapeDtypeStruct((b, d), jnp.float32),
        scratch_types=[
            pltpu.VMEM((b_per_w,), jnp.int32),
            pltpu.VMEM((b_per_w, d), jnp.float32),
            pltpu.SemaphoreType.DMA,
        ],
    )
    def gather_kernel(table_hbm, idx_hbm, out_hbm, idx_v, rows_v, sem):
        wid = lax.axis_index("s") * nc + lax.axis_index("c")
        base = wid * b_per_w
        pltpu.sync_copy(idx_hbm.at[pl.ds(base, b_per_w)], idx_v)
        pltpu.async_copy(table_hbm.at[idx_v], rows_v, sem).wait()
        pltpu.sync_copy(rows_v, out_hbm.at[pl.ds(base, b_per_w)])

    return gather_kernel(table, idx)


def _h_body(x_ref, w1_ref, b1_ref, he_ref, w2_ref, b2_ref, h_ref):
    dn = (((1,), (1,)), ((), ()))
    z1 = lax.dot_general(x_ref[...], w1_ref[...], dn,
                         preferred_element_type=jnp.float32)
    nv1 = jnp.tanh(_ALPHA * (z1 + b1_ref[...]))
    z2 = lax.dot_general(he_ref[...], w2_ref[...], dn,
                         preferred_element_type=jnp.float32)
    nv2 = jnp.tanh(_ALPHA * (z2 + b2_ref[...]))
    h0 = lax.dot_general(nv1, nv2, dn, preferred_element_type=jnp.float32)
    h_ref[...] = jnp.maximum(jnp.tanh(_ALPHA * h0), 0.0)


def _adj_topk_body(hb_ref, hall_ref, out_ref, work_ref):
    blk, b = out_ref.shape
    adj = lax.dot_general(hb_ref[...], hall_ref[...], (((1,), (1,)), ((), ())),
                          preferred_element_type=jnp.float32)
    work_ref[...] = adj
    out_ref[...] = jnp.zeros((blk, b), jnp.float32)
    colid = lax.broadcasted_iota(jnp.int32, (blk, b), 1)

    def pick_one(_, carry):
        w = work_ref[...]
        m = jnp.max(w, axis=1, keepdims=True)
        eq = w == m
        pos = jnp.min(jnp.where(eq, colid, b), axis=1, keepdims=True)
        sel = colid == pos
        out_ref[...] = jnp.where(sel, w, out_ref[...])
        work_ref[...] = jnp.where(sel, -1.0, w)
        return carry

    lax.fori_loop(0, _K, pick_one, 0)


def kernel(idx, embn, embhe, W1, b1, W2, b2):
    b = idx.shape[0]
    nhedges, hedim = embhe.shape
    ndim = embn.shape[1]

    nv1_raw = _gather_rows_sc(embn, idx.astype(jnp.int32))

    blk_h = 512
    H = pl.pallas_call(
        _h_body,
        grid=(b // blk_h,),
        in_specs=[
            pl.BlockSpec((blk_h, ndim), lambda i: (i, 0)),
            pl.BlockSpec((W1.shape[0], ndim), lambda i: (0, 0)),
            pl.BlockSpec((1, W1.shape[0]), lambda i: (0, 0)),
            pl.BlockSpec((nhedges, hedim), lambda i: (0, 0)),
            pl.BlockSpec((W2.shape[0], hedim), lambda i: (0, 0)),
            pl.BlockSpec((1, W2.shape[0]), lambda i: (0, 0)),
        ],
        out_specs=pl.BlockSpec((blk_h, nhedges), lambda i: (i, 0)),
        out_shape=jax.ShapeDtypeStruct((b, nhedges), jnp.float32),
    )(nv1_raw, W1, b1.reshape(1, -1), embhe, W2, b2.reshape(1, -1))

    blk_a = 256
    adj = pl.pallas_call(
        _adj_topk_body,
        grid=(b // blk_a,),
        in_specs=[
            pl.BlockSpec((blk_a, nhedges), lambda i: (i, 0)),
            pl.BlockSpec((b, nhedges), lambda i: (0, 0)),
        ],
        out_specs=pl.BlockSpec((blk_a, b), lambda i: (i, 0)),
        out_shape=jax.ShapeDtypeStruct((b, b), jnp.float32),
        scratch_shapes=[pltpu.VMEM((blk_a, b), jnp.float32)],
    )(H, H)

    return adj


# speedup vs baseline: 10.1127x; 1.3860x over previous
import functools

import jax
import jax.numpy as jnp
from jax import lax
from jax.experimental import pallas as pl
from jax.experimental.pallas import tpu as pltpu
from jax.experimental.pallas import tpu_sc as plsc

_ALPHA = 3.0
_K = 20


def _gather_rows_sc(table, idx):
    info = plsc.get_sparse_core_info()
    nc, ns = info.num_cores, info.num_subcores
    nw = nc * ns
    b, d = idx.shape[0], table.shape[1]
    b_per_w = b // nw
    mesh = plsc.VectorSubcoreMesh(core_axis_name="c", subcore_axis_name="s")

    @functools.partial(
        pl.kernel,
        mesh=mesh,
        compiler_params=pltpu.CompilerParams(use_tc_tiling_on_sc=False),
        out_type=jax.ShapeDtypeStruct((b, d), jnp.float32),
        scratch_types=[
            pltpu.VMEM((b_per_w,), jnp.int32),
            pltpu.VMEM((b_per_w, d), jnp.float32),
            pltpu.SemaphoreType.DMA,
        ],
    )
    def gather_kernel(table_hbm, idx_hbm, out_hbm, idx_v, rows_v, sem):
        wid = lax.axis_index("s") * nc + lax.axis_index("c")
        base = wid * b_per_w
        pltpu.sync_copy(idx_hbm.at[pl.ds(base, b_per_w)], idx_v)
        pltpu.async_copy(table_hbm.at[idx_v], rows_v, sem).wait()
        pltpu.sync_copy(rows_v, out_hbm.at[pl.ds(base, b_per_w)])

    return gather_kernel(table, idx)


def _h_body(x_ref, w1_ref, b1_ref, he_ref, w2_ref, b2_ref, h_ref):
    dn = (((1,), (1,)), ((), ()))
    z1 = lax.dot_general(x_ref[...], w1_ref[...], dn,
                         preferred_element_type=jnp.float32)
    nv1 = jnp.tanh(_ALPHA * (z1 + b1_ref[...]))
    z2 = lax.dot_general(he_ref[...], w2_ref[...], dn,
                         preferred_element_type=jnp.float32)
    nv2 = jnp.tanh(_ALPHA * (z2 + b2_ref[...]))
    h0 = lax.dot_general(nv1, nv2, dn, preferred_element_type=jnp.float32)
    h_ref[...] = jnp.maximum(jnp.tanh(_ALPHA * h0), 0.0)


def _adj_topk_body(hb_ref, hall_ref, out_ref, work_ref):
    blk, b = out_ref.shape
    adj = lax.dot_general(hb_ref[...], hall_ref[...], (((1,), (1,)), ((), ())),
                          preferred_element_type=jnp.float32)
    out_ref[...] = adj
    work_ref[...] = adj
    colid = lax.broadcasted_iota(jnp.int32, (blk, b), 1)

    def pick_one(_, carry):
        w = work_ref[...]
        m = jnp.max(w, axis=1, keepdims=True)
        pos = jnp.min(jnp.where(w == m, colid, b), axis=1, keepdims=True)
        work_ref[...] = jnp.where(colid == pos, -1.0, w)
        return carry

    lax.fori_loop(0, _K, pick_one, 0)
    out_ref[...] = jnp.where(work_ref[...] < 0.0, out_ref[...], 0.0)


def kernel(idx, embn, embhe, W1, b1, W2, b2):
    b = idx.shape[0]
    nhedges, hedim = embhe.shape
    ndim = embn.shape[1]

    nv1_raw = _gather_rows_sc(embn, idx.astype(jnp.int32))

    blk_h = 512
    H = pl.pallas_call(
        _h_body,
        grid=(b // blk_h,),
        in_specs=[
            pl.BlockSpec((blk_h, ndim), lambda i: (i, 0)),
            pl.BlockSpec((W1.shape[0], ndim), lambda i: (0, 0)),
            pl.BlockSpec((1, W1.shape[0]), lambda i: (0, 0)),
            pl.BlockSpec((nhedges, hedim), lambda i: (0, 0)),
            pl.BlockSpec((W2.shape[0], hedim), lambda i: (0, 0)),
            pl.BlockSpec((1, W2.shape[0]), lambda i: (0, 0)),
        ],
        out_specs=pl.BlockSpec((blk_h, nhedges), lambda i: (i, 0)),
        out_shape=jax.ShapeDtypeStruct((b, nhedges), jnp.float32),
        compiler_params=pltpu.CompilerParams(
            dimension_semantics=("parallel",)),
    )(nv1_raw, W1, b1.reshape(1, -1), embhe, W2, b2.reshape(1, -1))

    blk_a = 256
    adj = pl.pallas_call(
        _adj_topk_body,
        grid=(b // blk_a,),
        in_specs=[
            pl.BlockSpec((blk_a, nhedges), lambda i: (i, 0)),
            pl.BlockSpec((b, nhedges), lambda i: (0, 0)),
        ],
        out_specs=pl.BlockSpec((blk_a, b), lambda i: (i, 0)),
        out_shape=jax.ShapeDtypeStruct((b, b), jnp.float32),
        scratch_shapes=[pltpu.VMEM((blk_a, b), jnp.float32)],
        compiler_params=pltpu.CompilerParams(
            dimension_semantics=("parallel",)),
    )(H, H)

    return adj
